# baseline (speedup 1.0000x reference)
import jax
import jax.numpy as jnp
from jax import lax
from jax.experimental import pallas as pl
from jax.experimental.pallas import tpu as pltpu
from jax.experimental.pallas import tpu_sc as plsc

N_NODES = 100000
ROWS = 800
E = 6400000
NW = 32

CHUNK = 2048
TOTAL_CHUNKS = E // CHUNK

_MESH = plsc.VectorSubcoreMesh(core_axis_name="c", subcore_axis_name="s")
_SC_PARAMS = pltpu.CompilerParams(needs_layout_passes=False)


def _tile_chunks(wid):
    first = wid * 98 - jnp.maximum(wid - 21, 0)
    count = jnp.where(wid < 21, 98, 97)
    return first, count


def _kernel_sc(out_type, scratch):
    def deco(body):
        return pl.kernel(
            body,
            out_type=out_type,
            mesh=_MESH,
            scratch_types=scratch,
            compiler_params=_SC_PARAMS,
        )
    return deco


@_kernel_sc(
    jax.ShapeDtypeStruct((NW, ROWS, 128), jnp.float32),
    [
        pltpu.VMEM((2, 2 * CHUNK), jnp.int32),
        pltpu.VMEM((ROWS, 128), jnp.float32),
        pltpu.SemaphoreType.DMA,
    ],
)
def _hist_kernel(edge_hbm, out_hbm, edge_v, hist_v, sem_in):
    cid = lax.axis_index("c")
    sid = lax.axis_index("s")
    wid = cid * 16 + sid
    first, count = _tile_chunks(wid)
    ebase = first * CHUNK

    pltpu.async_copy(
        edge_hbm.at[:, pl.ds(ebase, CHUNK)], edge_v.at[:, pl.ds(0, CHUNK)], sem_in
    )

    zeros16 = jnp.zeros((16,), jnp.float32)

    @plsc.parallel_loop(0, ROWS, unroll=4)
    def _(i):
        for u in range(8):
            hist_v[i, pl.ds(u * 16, 16)] = zeros16

    ones16 = jnp.ones((16,), jnp.float32)

    def chunk_body(k, carry):
        b = lax.rem(k, 2)
        boff = b * CHUNK
        pltpu.make_async_copy(
            edge_hbm.at[:, pl.ds(ebase + k * CHUNK, CHUNK)],
            edge_v.at[:, pl.ds(boff, CHUNK)],
            sem_in,
        ).wait()

        @pl.when(k + 1 < count)
        def _():
            pltpu.async_copy(
                edge_hbm.at[:, pl.ds(ebase + (k + 1) * CHUNK, CHUNK)],
                edge_v.at[:, pl.ds((1 - b) * CHUNK, CHUNK)],
                sem_in,
            )

        @plsc.parallel_loop(0, CHUNK, 16, unroll=8)
        def _(g):
            x = edge_v[1, pl.ds(boff + g, 16)]
            plsc.addupdate_scatter(
                hist_v,
                [jnp.right_shift(x, 7), jnp.bitwise_and(x, 127)],
                ones16,
            )

        return carry

    lax.fori_loop(0, count, chunk_body, 0)
    pltpu.sync_copy(hist_v, out_hbm.at[wid])


def _reduce_body(h_ref, o_ref):
    deg = jnp.sum(h_ref[...], axis=0)
    o_ref[...] = jnp.where(deg > 0.0, lax.rsqrt(deg), 0.0)


def _deg_inv_sqrt(hist):
    return pl.pallas_call(
        _reduce_body,
        out_shape=jax.ShapeDtypeStruct((ROWS, 128), jnp.float32),
    )(hist)


@_kernel_sc(
    jax.ShapeDtypeStruct((E,), jnp.float32),
    [
        pltpu.VMEM((ROWS, 128), jnp.float32),
        pltpu.VMEM((2, 2 * CHUNK), jnp.int32),
        pltpu.VMEM((2 * CHUNK,), jnp.float32),
        pltpu.SemaphoreType.DMA,
        pltpu.SemaphoreType.DMA,
    ],
)
def _norm_kernel(edge_hbm, dis_hbm, out_hbm, tab_v, edge_v, out_v, sem_in, sem_o):
    cid = lax.axis_index("c")
    sid = lax.axis_index("s")
    wid = cid * 16 + sid
    first, count = _tile_chunks(wid)
    ebase = first * CHUNK

    pltpu.async_copy(
        edge_hbm.at[:, pl.ds(ebase, CHUNK)], edge_v.at[:, pl.ds(0, CHUNK)], sem_in
    )
    pltpu.sync_copy(dis_hbm, tab_v)

    def chunk_body(k, carry):
        b = lax.rem(k, 2)
        boff = b * CHUNK
        base = ebase + k * CHUNK
        pltpu.make_async_copy(
            edge_hbm.at[:, pl.ds(base, CHUNK)],
            edge_v.at[:, pl.ds(boff, CHUNK)],
            sem_in,
        ).wait()

        @pl.when(k + 1 < count)
        def _():
            pltpu.async_copy(
                edge_hbm.at[:, pl.ds(base + CHUNK, CHUNK)],
                edge_v.at[:, pl.ds((1 - b) * CHUNK, CHUNK)],
                sem_in,
            )

        @pl.when(k >= 2)
        def _():
            pltpu.make_async_copy(
                out_v.at[pl.ds(boff, CHUNK)], out_hbm.at[pl.ds(base, CHUNK)], sem_o
            ).wait()

        @plsc.parallel_loop(0, CHUNK, 16, unroll=8)
        def _(g):
            off = boff + g
            r = edge_v[0, pl.ds(off, 16)]
            c = edge_v[1, pl.ds(off, 16)]
            a = plsc.load_gather(
                tab_v, [jnp.right_shift(r, 7), jnp.bitwise_and(r, 127)]
            )
            bb = plsc.load_gather(
                tab_v, [jnp.right_shift(c, 7), jnp.bitwise_and(c, 127)]
            )
            out_v[pl.ds(off, 16)] = a * bb

        pltpu.async_copy(out_v.at[pl.ds(boff, CHUNK)], out_hbm.at[pl.ds(base, CHUNK)], sem_o)
        return carry

    lax.fori_loop(0, count, chunk_body, 0)
    last = ebase + (count - 1) * CHUNK
    pltpu.make_async_copy(
        out_v.at[pl.ds(0, CHUNK)], out_hbm.at[pl.ds(last, CHUNK)], sem_o
    ).wait()
    pltpu.make_async_copy(
        out_v.at[pl.ds(0, CHUNK)], out_hbm.at[pl.ds(last, CHUNK)], sem_o
    ).wait()


def kernel(edge_index, num_nodes):
    del num_nodes
    hist = _hist_kernel(edge_index)
    dis = _deg_inv_sqrt(hist)
    return _norm_kernel(edge_index, dis)


# speedup vs baseline: 1.0008x; 1.0008x over previous
import jax
import jax.numpy as jnp
from jax import lax
from jax.experimental import pallas as pl
from jax.experimental.pallas import tpu as pltpu
from jax.experimental.pallas import tpu_sc as plsc

N_NODES = 100000
N_PAD = 102400
E = 6400000
NW = 32

CHUNK = 2048
TOTAL_CHUNKS = E // CHUNK

_MESH = plsc.VectorSubcoreMesh(core_axis_name="c", subcore_axis_name="s")
_SC_PARAMS = pltpu.CompilerParams(needs_layout_passes=False)


def _tile_chunks(wid):
    first = wid * 98 - jnp.maximum(wid - 21, 0)
    count = jnp.where(wid < 21, 98, 97)
    return first, count


def _kernel_sc(out_type, scratch):
    def deco(body):
        return pl.kernel(
            body,
            out_type=out_type,
            mesh=_MESH,
            scratch_types=scratch,
            compiler_params=_SC_PARAMS,
        )
    return deco


@_kernel_sc(
    jax.ShapeDtypeStruct((NW * N_PAD,), jnp.float32),
    [
        pltpu.VMEM((2, 2 * CHUNK), jnp.int32),
        pltpu.VMEM((N_PAD,), jnp.float32),
        pltpu.SemaphoreType.DMA,
    ],
)
def _hist_kernel(edge_hbm, out_hbm, edge_v, hist_v, sem_in):
    cid = lax.axis_index("c")
    sid = lax.axis_index("s")
    wid = cid * 16 + sid
    first, count = _tile_chunks(wid)
    ebase = first * CHUNK

    pltpu.async_copy(
        edge_hbm.at[:, pl.ds(ebase, CHUNK)], edge_v.at[:, pl.ds(0, CHUNK)], sem_in
    )

    zeros16 = jnp.zeros((16,), jnp.float32)

    @plsc.parallel_loop(0, N_PAD, 16, unroll=8)
    def _(i):
        hist_v[pl.ds(i, 16)] = zeros16

    ones16 = jnp.ones((16,), jnp.float32)

    def chunk_body(k, carry):
        b = lax.rem(k, 2)
        boff = b * CHUNK
        pltpu.make_async_copy(
            edge_hbm.at[:, pl.ds(ebase + k * CHUNK, CHUNK)],
            edge_v.at[:, pl.ds(boff, CHUNK)],
            sem_in,
        ).wait()

        @pl.when(k + 1 < count)
        def _():
            pltpu.async_copy(
                edge_hbm.at[:, pl.ds(ebase + (k + 1) * CHUNK, CHUNK)],
                edge_v.at[:, pl.ds((1 - b) * CHUNK, CHUNK)],
                sem_in,
            )

        @plsc.parallel_loop(0, CHUNK, 16, unroll=8)
        def _(g):
            x = edge_v[1, pl.ds(boff + g, 16)]
            plsc.addupdate_scatter(hist_v, [x], ones16)

        return carry

    lax.fori_loop(0, count, chunk_body, 0)
    pltpu.sync_copy(hist_v, out_hbm.at[pl.ds(wid * N_PAD, N_PAD)])


def _reduce_body(h_ref, o_ref):
    deg = jnp.sum(h_ref[...], axis=0)
    o_ref[...] = jnp.where(deg > 0.0, lax.rsqrt(deg), 0.0)


def _deg_inv_sqrt(hist_flat):
    return pl.pallas_call(
        _reduce_body,
        out_shape=jax.ShapeDtypeStruct((N_PAD // 128, 128), jnp.float32),
    )(hist_flat.reshape(NW, N_PAD // 128, 128))


@_kernel_sc(
    jax.ShapeDtypeStruct((E,), jnp.float32),
    [
        pltpu.VMEM((N_PAD,), jnp.float32),
        pltpu.VMEM((2, 2 * CHUNK), jnp.int32),
        pltpu.VMEM((2 * CHUNK,), jnp.float32),
        pltpu.SemaphoreType.DMA,
        pltpu.SemaphoreType.DMA,
    ],
)
def _norm_kernel(edge_hbm, dis_hbm, out_hbm, tab_v, edge_v, out_v, sem_in, sem_o):
    cid = lax.axis_index("c")
    sid = lax.axis_index("s")
    wid = cid * 16 + sid
    first, count = _tile_chunks(wid)
    ebase = first * CHUNK

    pltpu.async_copy(
        edge_hbm.at[:, pl.ds(ebase, CHUNK)], edge_v.at[:, pl.ds(0, CHUNK)], sem_in
    )
    pltpu.sync_copy(dis_hbm, tab_v)

    def chunk_body(k, carry):
        b = lax.rem(k, 2)
        boff = b * CHUNK
        base = ebase + k * CHUNK
        pltpu.make_async_copy(
            edge_hbm.at[:, pl.ds(base, CHUNK)],
            edge_v.at[:, pl.ds(boff, CHUNK)],
            sem_in,
        ).wait()

        @pl.when(k + 1 < count)
        def _():
            pltpu.async_copy(
                edge_hbm.at[:, pl.ds(base + CHUNK, CHUNK)],
                edge_v.at[:, pl.ds((1 - b) * CHUNK, CHUNK)],
                sem_in,
            )

        @pl.when(k >= 2)
        def _():
            pltpu.make_async_copy(
                out_v.at[pl.ds(boff, CHUNK)], out_hbm.at[pl.ds(base, CHUNK)], sem_o
            ).wait()

        @plsc.parallel_loop(0, CHUNK, 16, unroll=8)
        def _(g):
            off = boff + g
            r = edge_v[0, pl.ds(off, 16)]
            c = edge_v[1, pl.ds(off, 16)]
            a = plsc.load_gather(tab_v, [r])
            bb = plsc.load_gather(tab_v, [c])
            out_v[pl.ds(off, 16)] = a * bb

        pltpu.async_copy(out_v.at[pl.ds(boff, CHUNK)], out_hbm.at[pl.ds(base, CHUNK)], sem_o)
        return carry

    lax.fori_loop(0, count, chunk_body, 0)
    last = ebase + (count - 1) * CHUNK
    pltpu.make_async_copy(
        out_v.at[pl.ds(0, CHUNK)], out_hbm.at[pl.ds(last, CHUNK)], sem_o
    ).wait()
    pltpu.make_async_copy(
        out_v.at[pl.ds(0, CHUNK)], out_hbm.at[pl.ds(last, CHUNK)], sem_o
    ).wait()


def kernel(edge_index, num_nodes):
    del num_nodes
    hist_flat = _hist_kernel(edge_index)
    dis = _deg_inv_sqrt(hist_flat).reshape(N_PAD)
    return _norm_kernel(edge_index, dis)


# speedup vs baseline: 1.8583x; 1.8567x over previous
import jax
import jax.numpy as jnp
from jax import lax
from jax.experimental import pallas as pl
from jax.experimental.pallas import tpu as pltpu
from jax.experimental.pallas import tpu_sc as plsc

N_NODES = 100000
N_PAD = 102400
E = 6400000
NW = 32

CHUNK = 2048
NBUF = 4
TOTAL_CHUNKS = E // CHUNK

_MESH = plsc.VectorSubcoreMesh(core_axis_name="c", subcore_axis_name="s")
_SC_PARAMS = pltpu.CompilerParams(needs_layout_passes=False)


def _tile_chunks(wid):
    first = wid * 98 - jnp.maximum(wid - 21, 0)
    count = jnp.where(wid < 21, 98, 97)
    return first, count


def _kernel_sc(out_type, scratch):
    def deco(body):
        return pl.kernel(
            body,
            out_type=out_type,
            mesh=_MESH,
            scratch_types=scratch,
            compiler_params=_SC_PARAMS,
        )
    return deco


@_kernel_sc(
    jax.ShapeDtypeStruct((NW * N_PAD,), jnp.float32),
    [
        pltpu.VMEM((2, NBUF * CHUNK), jnp.int32),
        pltpu.VMEM((N_PAD,), jnp.float32),
        pltpu.SemaphoreType.DMA,
    ],
)
def _hist_kernel(edge_hbm, out_hbm, edge_v, hist_v, sem_in):
    cid = lax.axis_index("c")
    sid = lax.axis_index("s")
    wid = cid * 16 + sid
    first, count = _tile_chunks(wid)
    ebase = first * CHUNK

    for p in range(NBUF):
        pltpu.async_copy(
            edge_hbm.at[:, pl.ds(ebase + p * CHUNK, CHUNK)],
            edge_v.at[:, pl.ds(p * CHUNK, CHUNK)],
            sem_in,
        )

    zeros16 = jnp.zeros((16,), jnp.float32)

    @plsc.parallel_loop(0, N_PAD, 16, unroll=8)
    def _(i):
        hist_v[pl.ds(i, 16)] = zeros16

    ones16 = jnp.ones((16,), jnp.float32)

    def chunk_body(k, carry):
        boff = lax.rem(k, NBUF) * CHUNK
        pltpu.make_async_copy(
            edge_hbm.at[:, pl.ds(ebase + k * CHUNK, CHUNK)],
            edge_v.at[:, pl.ds(boff, CHUNK)],
            sem_in,
        ).wait()

        @plsc.parallel_loop(0, CHUNK, 16, unroll=8)
        def _(g):
            x = edge_v[1, pl.ds(boff + g, 16)]
            plsc.addupdate_scatter(hist_v, [x], ones16)

        @pl.when(k + NBUF < count)
        def _():
            pltpu.async_copy(
                edge_hbm.at[:, pl.ds(ebase + (k + NBUF) * CHUNK, CHUNK)],
                edge_v.at[:, pl.ds(boff, CHUNK)],
                sem_in,
            )

        return carry

    lax.fori_loop(0, count, chunk_body, 0)
    pltpu.sync_copy(hist_v, out_hbm.at[pl.ds(wid * N_PAD, N_PAD)])


def _reduce_body(h_ref, o_ref):
    deg = jnp.sum(h_ref[...], axis=0)
    o_ref[...] = jnp.where(deg > 0.0, lax.rsqrt(deg), 0.0)


def _deg_inv_sqrt(hist_flat):
    return pl.pallas_call(
        _reduce_body,
        out_shape=jax.ShapeDtypeStruct((N_PAD // 128, 128), jnp.float32),
    )(hist_flat.reshape(NW, N_PAD // 128, 128))


@_kernel_sc(
    jax.ShapeDtypeStruct((E,), jnp.float32),
    [
        pltpu.VMEM((N_PAD,), jnp.float32),
        pltpu.VMEM((2, NBUF * CHUNK), jnp.int32),
        pltpu.VMEM((NBUF * CHUNK,), jnp.float32),
        pltpu.SemaphoreType.DMA,
        pltpu.SemaphoreType.DMA,
    ],
)
def _norm_kernel(edge_hbm, dis_hbm, out_hbm, tab_v, edge_v, out_v, sem_in, sem_o):
    cid = lax.axis_index("c")
    sid = lax.axis_index("s")
    wid = cid * 16 + sid
    first, count = _tile_chunks(wid)
    ebase = first * CHUNK

    for p in range(NBUF):
        pltpu.async_copy(
            edge_hbm.at[:, pl.ds(ebase + p * CHUNK, CHUNK)],
            edge_v.at[:, pl.ds(p * CHUNK, CHUNK)],
            sem_in,
        )
    pltpu.sync_copy(dis_hbm, tab_v)

    def chunk_body(k, carry):
        boff = lax.rem(k, NBUF) * CHUNK
        base = ebase + k * CHUNK
        pltpu.make_async_copy(
            edge_hbm.at[:, pl.ds(base, CHUNK)],
            edge_v.at[:, pl.ds(boff, CHUNK)],
            sem_in,
        ).wait()

        @pl.when(k >= NBUF)
        def _():
            pltpu.make_async_copy(
                out_v.at[pl.ds(boff, CHUNK)], out_hbm.at[pl.ds(base, CHUNK)], sem_o
            ).wait()

        @plsc.parallel_loop(0, CHUNK, 16, unroll=8)
        def _(g):
            off = boff + g
            r = edge_v[0, pl.ds(off, 16)]
            c = edge_v[1, pl.ds(off, 16)]
            a = plsc.load_gather(tab_v, [r])
            bb = plsc.load_gather(tab_v, [c])
            out_v[pl.ds(off, 16)] = a * bb

        pltpu.async_copy(out_v.at[pl.ds(boff, CHUNK)], out_hbm.at[pl.ds(base, CHUNK)], sem_o)

        @pl.when(k + NBUF < count)
        def _():
            pltpu.async_copy(
                edge_hbm.at[:, pl.ds(base + NBUF * CHUNK, CHUNK)],
                edge_v.at[:, pl.ds(boff, CHUNK)],
                sem_in,
            )

        return carry

    lax.fori_loop(0, count, chunk_body, 0)
    last = ebase + (count - 1) * CHUNK
    for _ in range(NBUF):
        pltpu.make_async_copy(
            out_v.at[pl.ds(0, CHUNK)], out_hbm.at[pl.ds(last, CHUNK)], sem_o
        ).wait()


def kernel(edge_index, num_nodes):
    del num_nodes
    hist_flat = _hist_kernel(edge_index)
    dis = _deg_inv_sqrt(hist_flat).reshape(N_PAD)
    return _norm_kernel(edge_index, dis)


# speedup vs baseline: 1.8641x; 1.0031x over previous
import jax
import jax.numpy as jnp
from jax import lax
from jax.experimental import pallas as pl
from jax.experimental.pallas import tpu as pltpu
from jax.experimental.pallas import tpu_sc as plsc

N_NODES = 100000
N_PAD = 102400
E = 6400000
NW = 32

CHUNK = 2048
NBUF = 4
TOTAL_CHUNKS = E // CHUNK

_MESH = plsc.VectorSubcoreMesh(core_axis_name="c", subcore_axis_name="s")
_SC_PARAMS = pltpu.CompilerParams(needs_layout_passes=False)


def _tile_chunks(wid):
    first = wid * 98 - jnp.maximum(wid - 21, 0)
    count = jnp.where(wid < 21, 98, 97)
    return first, count


def _kernel_sc(out_type, scratch):
    def deco(body):
        return pl.kernel(
            body,
            out_type=out_type,
            mesh=_MESH,
            scratch_types=scratch,
            compiler_params=_SC_PARAMS,
        )
    return deco


@_kernel_sc(
    jax.ShapeDtypeStruct((NW * N_PAD,), jnp.float32),
    [
        pltpu.VMEM((2, NBUF * CHUNK), jnp.int32),
        pltpu.VMEM((N_PAD,), jnp.float32),
        pltpu.SemaphoreType.DMA,
    ],
)
def _hist_kernel(edge_hbm, out_hbm, edge_v, hist_v, sem_in):
    cid = lax.axis_index("c")
    sid = lax.axis_index("s")
    wid = cid * 16 + sid
    first, count = _tile_chunks(wid)
    ebase = first * CHUNK

    for p in range(NBUF):
        pltpu.async_copy(
            edge_hbm.at[:, pl.ds(ebase + p * CHUNK, CHUNK)],
            edge_v.at[:, pl.ds(p * CHUNK, CHUNK)],
            sem_in,
        )

    zeros16 = jnp.zeros((16,), jnp.float32)

    @plsc.parallel_loop(0, N_PAD, 16, unroll=16)
    def _(i):
        hist_v[pl.ds(i, 16)] = zeros16

    ones16 = jnp.ones((16,), jnp.float32)

    def chunk_body(k, carry):
        boff = lax.rem(k, NBUF) * CHUNK
        pltpu.make_async_copy(
            edge_hbm.at[:, pl.ds(ebase + k * CHUNK, CHUNK)],
            edge_v.at[:, pl.ds(boff, CHUNK)],
            sem_in,
        ).wait()

        @plsc.parallel_loop(0, CHUNK, 16, unroll=16)
        def _(g):
            x = edge_v[1, pl.ds(boff + g, 16)]
            plsc.addupdate_scatter(hist_v, [x], ones16)

        @pl.when(k + NBUF < count)
        def _():
            pltpu.async_copy(
                edge_hbm.at[:, pl.ds(ebase + (k + NBUF) * CHUNK, CHUNK)],
                edge_v.at[:, pl.ds(boff, CHUNK)],
                sem_in,
            )

        return carry

    lax.fori_loop(0, count, chunk_body, 0)
    pltpu.sync_copy(hist_v, out_hbm.at[pl.ds(wid * N_PAD, N_PAD)])


def _reduce_body(h_ref, o_ref):
    deg = jnp.sum(h_ref[...], axis=0)
    o_ref[...] = jnp.where(deg > 0.0, lax.rsqrt(deg), 0.0)


def _deg_inv_sqrt(hist_flat):
    return pl.pallas_call(
        _reduce_body,
        out_shape=jax.ShapeDtypeStruct((N_PAD // 128, 128), jnp.float32),
    )(hist_flat.reshape(NW, N_PAD // 128, 128))


@_kernel_sc(
    jax.ShapeDtypeStruct((E,), jnp.float32),
    [
        pltpu.VMEM((N_PAD,), jnp.float32),
        pltpu.VMEM((2, NBUF * CHUNK), jnp.int32),
        pltpu.VMEM((NBUF * CHUNK,), jnp.float32),
        pltpu.SemaphoreType.DMA,
        pltpu.SemaphoreType.DMA,
    ],
)
def _norm_kernel(edge_hbm, dis_hbm, out_hbm, tab_v, edge_v, out_v, sem_in, sem_o):
    cid = lax.axis_index("c")
    sid = lax.axis_index("s")
    wid = cid * 16 + sid
    first, count = _tile_chunks(wid)
    ebase = first * CHUNK

    for p in range(NBUF):
        pltpu.async_copy(
            edge_hbm.at[:, pl.ds(ebase + p * CHUNK, CHUNK)],
            edge_v.at[:, pl.ds(p * CHUNK, CHUNK)],
            sem_in,
        )
    pltpu.sync_copy(dis_hbm, tab_v)

    def chunk_body(k, carry):
        boff = lax.rem(k, NBUF) * CHUNK
        base = ebase + k * CHUNK
        pltpu.make_async_copy(
            edge_hbm.at[:, pl.ds(base, CHUNK)],
            edge_v.at[:, pl.ds(boff, CHUNK)],
            sem_in,
        ).wait()

        @pl.when(k >= NBUF)
        def _():
            pltpu.make_async_copy(
                out_v.at[pl.ds(boff, CHUNK)], out_hbm.at[pl.ds(base, CHUNK)], sem_o
            ).wait()

        @plsc.parallel_loop(0, CHUNK, 16, unroll=16)
        def _(g):
            off = boff + g
            r = edge_v[0, pl.ds(off, 16)]
            c = edge_v[1, pl.ds(off, 16)]
            a = plsc.load_gather(tab_v, [r])
            bb = plsc.load_gather(tab_v, [c])
            out_v[pl.ds(off, 16)] = a * bb

        pltpu.async_copy(out_v.at[pl.ds(boff, CHUNK)], out_hbm.at[pl.ds(base, CHUNK)], sem_o)

        @pl.when(k + NBUF < count)
        def _():
            pltpu.async_copy(
                edge_hbm.at[:, pl.ds(base + NBUF * CHUNK, CHUNK)],
                edge_v.at[:, pl.ds(boff, CHUNK)],
                sem_in,
            )

        return carry

    lax.fori_loop(0, count, chunk_body, 0)
    last = ebase + (count - 1) * CHUNK
    for _ in range(NBUF):
        pltpu.make_async_copy(
            out_v.at[pl.ds(0, CHUNK)], out_hbm.at[pl.ds(last, CHUNK)], sem_o
        ).wait()


def kernel(edge_index, num_nodes):
    del num_nodes
    hist_flat = _hist_kernel(edge_index)
    dis = _deg_inv_sqrt(hist_flat).reshape(N_PAD)
    return _norm_kernel(edge_index, dis)
